# speedup vs baseline: 3.8329x; 3.8329x over previous
import jax
import jax.numpy as jnp
from jax import lax
from jax.experimental import pallas as pl
from jax.experimental.pallas import tpu as pltpu

_MIN_PIXELS = 128
_RATIO = 16
_H = 32
_HW = _H * _H


def _pool_body(mask_ref, sel_ref):
    bs = mask_ref.shape[0]
    r = lax.broadcasted_iota(jnp.int32, (_H, 512), 0)
    c = lax.broadcasted_iota(jnp.int32, (_H, 512), 1)
    pool = (c // _RATIO == r).astype(jnp.float32)
    for b in range(bs):
        m = mask_ref[b].astype(jnp.float32)
        t = jnp.dot(pool, m, preferred_element_type=jnp.float32)
        cnt = lax.dot_general(t, pool, (((1,), (1,)), ((), ())),
                              preferred_element_type=jnp.float32)
        sel_ref[b] = (cnt >= _MIN_PIXELS).astype(jnp.float32)


def _mean_body(ids_ref, sel_ref, emb_ref, out_ref):
    i = pl.program_id(0)
    n = pl.num_programs(0)
    belong = (ids_ref[...] == i).astype(jnp.float32)
    w = sel_ref[...] * belong
    contrib = lax.dot_general(w, emb_ref[0], (((1,), (1,)), ((), ())),
                              preferred_element_type=jnp.float32)

    @pl.when(i == 0)
    def _init():
        out_ref[...] = contrib

    @pl.when(i > 0)
    def _acc():
        out_ref[...] += contrib

    @pl.when(i == n - 1)
    def _final():
        den = jnp.sum(sel_ref[...], axis=1, keepdims=True)
        out_ref[...] = out_ref[...] / den


def kernel(binary_masks, image_ids, relative_segment_ids, coords,
           sam_encoder_embeddings):
    S = binary_masks.shape[0]
    n_envs = sam_encoder_embeddings.shape[0]
    C = sam_encoder_embeddings.shape[2]
    masks = binary_masks.reshape(S, 512, 512)
    emb = sam_encoder_embeddings.reshape(n_envs, C, _HW)

    BS = 4
    sel = pl.pallas_call(
        _pool_body,
        grid=(S // BS,),
        in_specs=[pl.BlockSpec((BS, 512, 512), lambda i: (i, 0, 0))],
        out_specs=pl.BlockSpec((BS, _H, _H), lambda i: (i, 0, 0)),
        out_shape=jax.ShapeDtypeStruct((S, _H, _H), jnp.float32),
    )(masks)

    sel2 = sel.reshape(S, _HW)
    ids2 = image_ids.reshape(S, 1)
    segs = pl.pallas_call(
        _mean_body,
        grid=(n_envs,),
        in_specs=[
            pl.BlockSpec((S, 1), lambda i: (0, 0)),
            pl.BlockSpec((S, _HW), lambda i: (0, 0)),
            pl.BlockSpec((1, C, _HW), lambda i: (i, 0, 0)),
        ],
        out_specs=pl.BlockSpec((S, C), lambda i: (0, 0)),
        out_shape=jax.ShapeDtypeStruct((S, C), jnp.float32),
    )(ids2, sel2, emb)

    is_latent_tokens = jnp.zeros((S,), dtype=bool)
    return (image_ids, relative_segment_ids, is_latent_tokens, segs, coords)
